# baseline (speedup 1.0000x reference)
import numpy as np
import jax
import jax.numpy as jnp
from jax.experimental import pallas as pl
from jax.experimental.pallas import tpu as pltpu


def _gdm_kernel(vf_ref, sfg_ref, We1t_ref, We2x_ref, We3_ref,
                It_ref, bet_ref, Wax_ref, ba_ref, E8_ref, Wn1_ref, Wn2_ref,
                bn_ref, Wp1_ref, Wp2x_ref, Wp3_ref, bp_ref, out_ref):
    gpb = sfg_ref.shape[0]
    k = vf_ref.shape[0] // gpb
    for t in range(gpb):
        _one_graph(vf_ref[t * k:(t + 1) * k, :], sfg_ref[t], We1t_ref,
                   We2x_ref, We3_ref, It_ref, bet_ref, Wax_ref, ba_ref,
                   E8_ref, Wn1_ref, Wn2_ref, bn_ref, Wp1_ref, Wp2x_ref,
                   Wp3_ref, bp_ref, out_ref, t)


def _one_graph(vf, sfg, We1t_ref, We2x_ref, We3_ref,
               It_ref, bet_ref, Wax_ref, ba_ref, E8_ref, Wn1_ref, Wn2_ref,
               bn_ref, Wp1_ref, Wp2x_ref, Wp3_ref, bp_ref, out_ref, t):
    k, dn = vf.shape
    g = We2x_ref.shape[1] // dn
    u8 = jnp.dot(vf, We1t_ref[...], preferred_element_type=jnp.float32) + bet_ref[...]
    v8 = jnp.dot(vf, We3_ref[...], preferred_element_type=jnp.float32).reshape(g, g * dn)
    vt8 = jnp.dot(vf, It_ref[...], preferred_element_type=jnp.float32)
    sw = jnp.dot(sfg, We2x_ref[...], preferred_element_type=jnp.float32)
    ef = jax.nn.relu(sw.reshape(k, g, g * dn) + u8[:, None, :] + v8[None, :, :])
    a2 = jax.nn.relu(jnp.dot(ef.reshape(k * g, g * dn), Wax_ref[...],
                             preferred_element_type=jnp.float32) + ba_ref[...])
    a3 = a2.reshape(k, g, g)
    i_s = jax.lax.broadcasted_iota(jnp.int32, (k, g, g), 0)
    i_a = jax.lax.broadcasted_iota(jnp.int32, (k, g, g), 1)
    i_b = jax.lax.broadcasted_iota(jnp.int32, (k, g, g), 2)
    a3 = jnp.where(g * i_a + i_b == i_s, jnp.float32(-1e30), a3)
    m = jnp.max(a3, axis=0, keepdims=True)
    ex = jnp.exp(a3 - m)
    den = jnp.sum(ex, axis=0, keepdims=True)
    alpha = ex / den
    alphaw = jnp.dot(alpha.reshape(k * g, g), E8_ref[...],
                     preferred_element_type=jnp.float32)
    zg = jnp.sum(alphaw.reshape(k, g, g * dn) * (vt8[:, None, :] + ef), axis=0)
    z = zg.reshape(k, dn)
    nn = jax.nn.relu(jnp.dot(vf, Wn1_ref[...], preferred_element_type=jnp.float32)
                     + jnp.dot(z, Wn2_ref[...], preferred_element_type=jnp.float32)
                     + bn_ref[...])
    dp = Wp3_ref.shape[1]
    spg = jnp.dot(sfg[0:k // g, :], Wp2x_ref[...],
                  preferred_element_type=jnp.float32)
    spd = spg.reshape(k, dn)
    pred = (jnp.dot(nn[0:1, :], Wp1_ref[...], preferred_element_type=jnp.float32)
            + spd[1:k, 0:dp]
            + jnp.dot(nn[1:k, :], Wp3_ref[...], preferred_element_type=jnp.float32)
            + bp_ref[...])
    out_ref[t] = pred


def kernel(node_num_list, visual_feat, spatial_feat, We, be, Wa, ba, Wn, bn,
           Wp, bp):
    b = node_num_list.shape[0]
    n, dn = visual_feat.shape
    k = n // b
    ds = spatial_feat.shape[1]
    dp = Wp.shape[1]
    g = dn // ds
    kg = k // g
    We1, We2, We3 = We[:dn], We[dn:dn + ds], We[dn + ds:]
    Wn1, Wn2 = Wn[:dn], Wn[dn:]
    Wp1, Wp2, Wp3 = Wp[:dn], Wp[dn:dn + ds], Wp[dn + ds:]

    We1t = jnp.tile(We1, (1, g))
    bet = jnp.tile(be.reshape(1, dn), (1, g))
    It = jnp.tile(jnp.eye(dn, dtype=jnp.float32), (1, g))
    We2x = jnp.asarray(np.kron(np.eye(g, dtype=np.float32), np.ones((ds, dn), np.float32))) \
        * jnp.tile(We2, (g, g))
    Wax = jnp.asarray(np.kron(np.eye(g, dtype=np.float32), np.ones((dn, 1), np.float32))) \
        * jnp.tile(Wa, (g, g))
    E8 = jnp.asarray(np.kron(np.eye(g, dtype=np.float32), np.ones((1, dn), np.float32)))
    Wp2p = jnp.pad(Wp2, ((0, 0), (0, dn - dp)))
    Wp2x = jnp.asarray(np.kron(np.eye(g, dtype=np.float32), np.ones((ds, dn), np.float32))) \
        * jnp.tile(Wp2p, (g, g))

    sf4 = spatial_feat.reshape(b, k - 1, k, ds)
    zc = jnp.zeros((b, k - 1, 1, ds), jnp.float32)
    sfd = jnp.concatenate([zc, sf4], axis=2).reshape(b, k * k - 1, ds)
    sfd = jnp.concatenate([sfd, jnp.zeros((b, 1, ds), jnp.float32)], axis=1)
    sfg = sfd.reshape(b, k * kg, g * ds)

    def const(*shape):
        return pl.BlockSpec(shape, lambda i: tuple(0 for _ in shape))

    gpb = 4
    out = pl.pallas_call(
        _gdm_kernel,
        grid=(b // gpb,),
        in_specs=[
            pl.BlockSpec((gpb * k, dn), lambda i: (i, 0)),
            pl.BlockSpec((gpb, k * kg, g * ds), lambda i: (i, 0, 0)),
            const(dn, g * dn), const(g * ds, g * dn), const(dn, dn),
            const(dn, g * dn), const(1, g * dn), const(g * dn, g),
            const(1, 1), const(g, g * dn),
            const(dn, dn), const(dn, dn), const(1, dn),
            const(dn, dp), const(g * ds, g * dn), const(dn, dp), const(1, dp),
        ],
        out_specs=pl.BlockSpec((gpb, k - 1, dp), lambda i: (i, 0, 0)),
        out_shape=jax.ShapeDtypeStruct((b, k - 1, dp), jnp.float32),
        compiler_params=pltpu.CompilerParams(
            dimension_semantics=("parallel",)),
    )(visual_feat, sfg, We1t, We2x, We3, It, bet, Wax,
      ba.reshape(1, 1), E8, Wn1, Wn2, bn.reshape(1, dn),
      Wp1, Wp2x, Wp3, bp.reshape(1, dp))
    return out.reshape(b * (k - 1), dp)


# speedup vs baseline: 1.0089x; 1.0089x over previous
import numpy as np
import jax
import jax.numpy as jnp
from jax.experimental import pallas as pl
from jax.experimental.pallas import tpu as pltpu


def _gdm_kernel(vf_ref, sfg_ref, We1t_ref, We2x_ref, We3_ref,
                It_ref, bet_ref, Wax_ref, ba_ref, E8_ref, Wn1_ref, Wn2_ref,
                bn_ref, Wp1_ref, Wp2x_ref, Wp3_ref, bp_ref, out_ref):
    gpb = sfg_ref.shape[0]
    k = vf_ref.shape[0] // gpb
    for t in range(gpb):
        _one_graph(vf_ref[t * k:(t + 1) * k, :], sfg_ref[t], We1t_ref,
                   We2x_ref, We3_ref, It_ref, bet_ref, Wax_ref, ba_ref,
                   E8_ref, Wn1_ref, Wn2_ref, bn_ref, Wp1_ref, Wp2x_ref,
                   Wp3_ref, bp_ref, out_ref, t)


def _one_graph(vf, sfg, We1t_ref, We2x_ref, We3_ref,
               It_ref, bet_ref, Wax_ref, ba_ref, E8_ref, Wn1_ref, Wn2_ref,
               bn_ref, Wp1_ref, Wp2x_ref, Wp3_ref, bp_ref, out_ref, t):
    k, dn = vf.shape
    g = We2x_ref.shape[1] // dn
    u8 = jnp.dot(vf, We1t_ref[...], preferred_element_type=jnp.float32) + bet_ref[...]
    v8 = jnp.dot(vf, We3_ref[...], preferred_element_type=jnp.float32).reshape(g, g * dn)
    vt8 = jnp.dot(vf, It_ref[...], preferred_element_type=jnp.float32)
    sw = jnp.dot(sfg, We2x_ref[...], preferred_element_type=jnp.float32)
    ef = jax.nn.relu(sw.reshape(k, g, g * dn) + u8[:, None, :] + v8[None, :, :])
    a2 = jax.nn.relu(jnp.dot(ef.reshape(k * g, g * dn), Wax_ref[...],
                             preferred_element_type=jnp.float32) + ba_ref[...])
    a3 = a2.reshape(k, g, g)
    i_s = jax.lax.broadcasted_iota(jnp.int32, (k, g, g), 0)
    i_a = jax.lax.broadcasted_iota(jnp.int32, (k, g, g), 1)
    i_b = jax.lax.broadcasted_iota(jnp.int32, (k, g, g), 2)
    a3 = jnp.where(g * i_a + i_b == i_s, jnp.float32(-1e30), a3)
    m = jnp.max(a3, axis=0, keepdims=True)
    ex = jnp.exp(a3 - m)
    den = jnp.sum(ex, axis=0, keepdims=True)
    alpha = ex / den
    alphaw = jnp.dot(alpha.reshape(k * g, g), E8_ref[...],
                     preferred_element_type=jnp.float32)
    zg = jnp.sum(alphaw.reshape(k, g, g * dn) * (vt8[:, None, :] + ef), axis=0)
    z = zg.reshape(k, dn)
    nn = jax.nn.relu(jnp.dot(vf, Wn1_ref[...], preferred_element_type=jnp.float32)
                     + jnp.dot(z, Wn2_ref[...], preferred_element_type=jnp.float32)
                     + bn_ref[...])
    dp = Wp3_ref.shape[1]
    spg = jnp.dot(sfg[0:k // g, :], Wp2x_ref[...],
                  preferred_element_type=jnp.float32)
    spd = spg.reshape(k, dn)
    pred = (jnp.dot(nn[0:1, :], Wp1_ref[...], preferred_element_type=jnp.float32)
            + spd[1:k, 0:dp]
            + jnp.dot(nn[1:k, :], Wp3_ref[...], preferred_element_type=jnp.float32)
            + bp_ref[...])
    out_ref[t] = pred


def kernel(node_num_list, visual_feat, spatial_feat, We, be, Wa, ba, Wn, bn,
           Wp, bp):
    b = node_num_list.shape[0]
    n, dn = visual_feat.shape
    k = n // b
    ds = spatial_feat.shape[1]
    dp = Wp.shape[1]
    g = dn // ds
    kg = k // g
    We1, We2, We3 = We[:dn], We[dn:dn + ds], We[dn + ds:]
    Wn1, Wn2 = Wn[:dn], Wn[dn:]
    Wp1, Wp2, Wp3 = Wp[:dn], Wp[dn:dn + ds], Wp[dn + ds:]

    We1t = jnp.tile(We1, (1, g))
    bet = jnp.tile(be.reshape(1, dn), (1, g))
    It = jnp.tile(jnp.eye(dn, dtype=jnp.float32), (1, g))
    We2x = jnp.asarray(np.kron(np.eye(g, dtype=np.float32), np.ones((ds, dn), np.float32))) \
        * jnp.tile(We2, (g, g))
    Wax = jnp.asarray(np.kron(np.eye(g, dtype=np.float32), np.ones((dn, 1), np.float32))) \
        * jnp.tile(Wa, (g, g))
    E8 = jnp.asarray(np.kron(np.eye(g, dtype=np.float32), np.ones((1, dn), np.float32)))
    Wp2p = jnp.pad(Wp2, ((0, 0), (0, dn - dp)))
    Wp2x = jnp.asarray(np.kron(np.eye(g, dtype=np.float32), np.ones((ds, dn), np.float32))) \
        * jnp.tile(Wp2p, (g, g))

    sf4 = spatial_feat.reshape(b, k - 1, k, ds)
    zc = jnp.zeros((b, k - 1, 1, ds), jnp.float32)
    sfd = jnp.concatenate([zc, sf4], axis=2).reshape(b, k * k - 1, ds)
    sfd = jnp.concatenate([sfd, jnp.zeros((b, 1, ds), jnp.float32)], axis=1)
    sfg = sfd.reshape(b, k * kg, g * ds)

    def const(*shape):
        return pl.BlockSpec(shape, lambda i: tuple(0 for _ in shape))

    gpb = 8
    out = pl.pallas_call(
        _gdm_kernel,
        grid=(b // gpb,),
        in_specs=[
            pl.BlockSpec((gpb * k, dn), lambda i: (i, 0)),
            pl.BlockSpec((gpb, k * kg, g * ds), lambda i: (i, 0, 0)),
            const(dn, g * dn), const(g * ds, g * dn), const(dn, dn),
            const(dn, g * dn), const(1, g * dn), const(g * dn, g),
            const(1, 1), const(g, g * dn),
            const(dn, dn), const(dn, dn), const(1, dn),
            const(dn, dp), const(g * ds, g * dn), const(dn, dp), const(1, dp),
        ],
        out_specs=pl.BlockSpec((gpb, k - 1, dp), lambda i: (i, 0, 0)),
        out_shape=jax.ShapeDtypeStruct((b, k - 1, dp), jnp.float32),
        compiler_params=pltpu.CompilerParams(
            dimension_semantics=("parallel",)),
    )(visual_feat, sfg, We1t, We2x, We3, It, bet, Wax,
      ba.reshape(1, 1), E8, Wn1, Wn2, bn.reshape(1, dn),
      Wp1, Wp2x, Wp3, bp.reshape(1, dp))
    return out.reshape(b * (k - 1), dp)
